# Initial kernel scaffold; baseline (speedup 1.0000x reference)
import jax
import jax.numpy as jnp
from jax.experimental import pallas as pl


def kernel(data, segments, num_segments, ctx):
    raise NotImplementedError("write your pallas kernel here")


# speedup vs baseline: 4.2173x; 4.2173x over previous
import functools

import jax
import jax.numpy as jnp
from jax import lax
from jax.experimental import pallas as pl
from jax.experimental.pallas import tpu as pltpu
from jax.experimental.pallas import tpu_sc as plsc

N_EDGES = 320000
D_FEAT = 128
N_SEGMENTS = 10000

_NC = 2
_NS = 16
_NW = _NC * _NS
_E_PER_TILE = N_EDGES // _NW
_CH = 40
_NCH = _E_PER_TILE // _CH
_HALF = _NCH // 2
_ROWS_PER_SUB = 624
_ROWS_LAST = N_SEGMENTS - 15 * _ROWS_PER_SUB
_ZROWS = 208


def _sc_body(data_hbm, seg_hbm, out_hbm,
             d0, d1, i0, i1, zb, acc, sd0, si0, sd1, si1):
    c = lax.axis_index("c")
    s = lax.axis_index("s")
    base = (c * _NS + s) * _E_PER_TILE

    def zrow(r, carry):
        def zcol(j, carry2):
            zb[r, pl.ds(j * 16, 16)] = jnp.zeros((16,), jnp.float32)
            return carry2
        return lax.fori_loop(0, D_FEAT // 16, zcol, carry)
    lax.fori_loop(0, _ZROWS, zrow, 0)

    def zcopy(k, carry):
        pltpu.sync_copy(zb, acc.at[pl.ds(s * _ROWS_PER_SUB + k * _ZROWS,
                                         _ZROWS), :])
        return carry
    lax.fori_loop(0, _ROWS_PER_SUB // _ZROWS, zcopy, 0)

    @pl.when(s == _NS - 1)
    def _():
        pltpu.sync_copy(zb.at[pl.ds(0, _ROWS_LAST - 3 * _ZROWS), :],
                        acc.at[pl.ds(15 * _ROWS_PER_SUB + 3 * _ZROWS,
                                     _ROWS_LAST - 3 * _ZROWS), :])
    plsc.subcore_barrier()

    def start(chunk, dbuf, ibuf, semd, semi):
        row = base + chunk * _CH
        pltpu.make_async_copy(data_hbm.at[pl.ds(row, _CH), :], dbuf,
                              semd).start()
        pltpu.make_async_copy(seg_hbm.at[pl.ds(row, _CH)], ibuf, semi).start()

    def wait(dbuf, ibuf, semd, semi):
        pltpu.make_async_copy(data_hbm.at[pl.ds(base, _CH), :], dbuf,
                              semd).wait()
        pltpu.make_async_copy(seg_hbm.at[pl.ds(base, _CH)], ibuf, semi).wait()

    start(0, d0, i0, sd0, si0)

    def body(g, carry):
        wait(d0, i0, sd0, si0)
        start(2 * g + 1, d1, i1, sd1, si1)
        pltpu.sync_copy(d0, acc.at[i0], add=True)

        wait(d1, i1, sd1, si1)

        @pl.when(2 * g + 2 < _NCH)
        def _():
            start(2 * g + 2, d0, i0, sd0, si0)

        pltpu.sync_copy(d1, acc.at[i1], add=True)
        return carry
    lax.fori_loop(0, _HALF, body, 0)
    plsc.subcore_barrier()

    r0 = s * _ROWS_PER_SUB

    @pl.when(s < _NS - 1)
    def _():
        pltpu.sync_copy(acc.at[pl.ds(r0, _ROWS_PER_SUB), :],
                        out_hbm.at[c, pl.ds(r0, _ROWS_PER_SUB), :])

    @pl.when(s == _NS - 1)
    def _():
        pltpu.sync_copy(acc.at[pl.ds(15 * _ROWS_PER_SUB, _ROWS_LAST), :],
                        out_hbm.at[c, pl.ds(15 * _ROWS_PER_SUB,
                                            _ROWS_LAST), :])


_sc_seg_sum = functools.partial(
    pl.kernel,
    out_type=jax.ShapeDtypeStruct((_NC, N_SEGMENTS, D_FEAT), jnp.float32),
    mesh=plsc.VectorSubcoreMesh(core_axis_name="c", subcore_axis_name="s"),
    scratch_types=[
        pltpu.VMEM((_CH, D_FEAT), jnp.float32),
        pltpu.VMEM((_CH, D_FEAT), jnp.float32),
        pltpu.VMEM((_CH,), jnp.int32),
        pltpu.VMEM((_CH,), jnp.int32),
        pltpu.VMEM((_ZROWS, D_FEAT), jnp.float32),
        pltpu.VMEM_SHARED((N_SEGMENTS, D_FEAT), jnp.float32),
        pltpu.SemaphoreType.DMA,
        pltpu.SemaphoreType.DMA,
        pltpu.SemaphoreType.DMA,
        pltpu.SemaphoreType.DMA,
    ],
)(_sc_body)


def _add_body(a_ref, b_ref, o_ref):
    o_ref[...] = a_ref[0] + b_ref[0]


_ROWS_PER_BLK = 1000


def _merge_partials(partial):
    return pl.pallas_call(
        _add_body,
        grid=(N_SEGMENTS // _ROWS_PER_BLK,),
        in_specs=[
            pl.BlockSpec((1, _ROWS_PER_BLK, D_FEAT), lambda i: (0, i, 0)),
            pl.BlockSpec((1, _ROWS_PER_BLK, D_FEAT), lambda i: (1, i, 0)),
        ],
        out_specs=pl.BlockSpec((_ROWS_PER_BLK, D_FEAT), lambda i: (i, 0)),
        out_shape=jax.ShapeDtypeStruct((N_SEGMENTS, D_FEAT), jnp.float32),
    )(partial, partial)


def kernel(data, segments, num_segments, ctx):
    partial = _sc_seg_sum(data, segments.astype(jnp.int32))
    return _merge_partials(partial)


# speedup vs baseline: 6.1164x; 1.4503x over previous
import functools

import jax
import jax.numpy as jnp
from jax import lax
from jax.experimental import pallas as pl
from jax.experimental.pallas import tpu as pltpu
from jax.experimental.pallas import tpu_sc as plsc

N_EDGES = 320000
D_FEAT = 128
N_SEGMENTS = 10000

_NC = 2
_NS = 16
_NW = _NC * _NS
_E_PER_TILE = N_EDGES // _NW
_CH = 80
_NB = 4
_NCH = _E_PER_TILE // _CH
_NGRP = (_NCH - 1) // _NB

_ROWS_PER_SUB = 624
_ROWS_LAST = N_SEGMENTS - 15 * _ROWS_PER_SUB


def _sc_body(data_hbm, seg_hbm, out_hbm,
             dbufs, ibufs, acc, sld, sli, ssc):
    c = lax.axis_index("c")
    s = lax.axis_index("s")
    base = (c * _NS + s) * _E_PER_TILE

    zb = dbufs[0]

    def zrow(r, carry):
        def zcol(j, carry2):
            zb[r, pl.ds(j * 16, 16)] = jnp.zeros((16,), jnp.float32)
            return carry2
        return lax.fori_loop(0, D_FEAT // 16, zcol, carry)
    lax.fori_loop(0, _CH, zrow, 0)

    def zcopy(k, carry):
        pltpu.sync_copy(zb, acc.at[pl.ds(s * _ROWS_PER_SUB + k * _CH,
                                         _CH), :])
        return carry
    lax.fori_loop(0, _ROWS_PER_SUB // _CH, zcopy, 0)

    pltpu.sync_copy(zb.at[pl.ds(0, _ROWS_PER_SUB - 7 * _CH), :],
                    acc.at[pl.ds(s * _ROWS_PER_SUB + 7 * _CH,
                                 _ROWS_PER_SUB - 7 * _CH), :])

    @pl.when(s == _NS - 1)
    def _():
        pltpu.sync_copy(zb.at[pl.ds(0, _ROWS_LAST - _ROWS_PER_SUB), :],
                        acc.at[pl.ds(15 * _ROWS_PER_SUB + _ROWS_PER_SUB,
                                     _ROWS_LAST - _ROWS_PER_SUB), :])
    plsc.subcore_barrier()

    def process_group(first_chunk, nb):
        loads = []
        for b in range(nb):
            row = base + (first_chunk + b) * _CH
            loads.append(pltpu.make_async_copy(
                data_hbm.at[pl.ds(row, _CH), :], dbufs[b], sld))
            loads.append(pltpu.make_async_copy(
                seg_hbm.at[pl.ds(row, _CH)], ibufs[b], sli))
        for h in loads:
            h.start()
        for h in loads:
            h.wait()
        scats = [pltpu.async_copy(dbufs[b], acc.at[ibufs[b]], ssc,
                                  add=True) for b in range(nb)]
        for h in scats:
            h.wait()

    def group(g, carry):
        process_group(_NB * g, _NB)
        return carry
    lax.fori_loop(0, _NGRP, group, 0)

    process_group(_NGRP * _NB, 1)
    plsc.subcore_barrier()

    r0 = s * _ROWS_PER_SUB

    @pl.when(s < _NS - 1)
    def _():
        pltpu.sync_copy(acc.at[pl.ds(r0, _ROWS_PER_SUB), :],
                        out_hbm.at[c, pl.ds(r0, _ROWS_PER_SUB), :])

    @pl.when(s == _NS - 1)
    def _():
        pltpu.sync_copy(acc.at[pl.ds(15 * _ROWS_PER_SUB, _ROWS_LAST), :],
                        out_hbm.at[c, pl.ds(15 * _ROWS_PER_SUB,
                                            _ROWS_LAST), :])


_sc_seg_sum = functools.partial(
    pl.kernel,
    out_type=jax.ShapeDtypeStruct((_NC, N_SEGMENTS, D_FEAT), jnp.float32),
    mesh=plsc.VectorSubcoreMesh(core_axis_name="c", subcore_axis_name="s"),
    scratch_types=[
        [pltpu.VMEM((_CH, D_FEAT), jnp.float32) for _ in range(_NB)],
        [pltpu.VMEM((_CH,), jnp.int32) for _ in range(_NB)],
        pltpu.VMEM_SHARED((N_SEGMENTS, D_FEAT), jnp.float32),
        pltpu.SemaphoreType.DMA,
        pltpu.SemaphoreType.DMA,
        pltpu.SemaphoreType.DMA,
    ],
)(_sc_body)


def _add_body(a_ref, b_ref, o_ref):
    o_ref[...] = a_ref[0] + b_ref[0]


_ROWS_PER_BLK = 1000


def _merge_partials(partial):
    return pl.pallas_call(
        _add_body,
        grid=(N_SEGMENTS // _ROWS_PER_BLK,),
        in_specs=[
            pl.BlockSpec((1, _ROWS_PER_BLK, D_FEAT), lambda i: (0, i, 0)),
            pl.BlockSpec((1, _ROWS_PER_BLK, D_FEAT), lambda i: (1, i, 0)),
        ],
        out_specs=pl.BlockSpec((_ROWS_PER_BLK, D_FEAT), lambda i: (i, 0)),
        out_shape=jax.ShapeDtypeStruct((N_SEGMENTS, D_FEAT), jnp.float32),
    )(partial, partial)


def kernel(data, segments, num_segments, ctx):
    partial = _sc_seg_sum(data, segments.astype(jnp.int32))
    return _merge_partials(partial)
